# baseline (speedup 1.0000x reference)
import functools

import jax
import jax.numpy as jnp
from jax import lax
from jax.experimental import pallas as pl
from jax.experimental.pallas import tpu as pltpu
from jax.experimental.pallas import tpu_sc as plsc

EMB_K = 64
N_WORKERS = 32


def _tr_body(src_ref, dst_ref):
    dst_ref[...] = src_ref[...].T


def _to_row_major(tabT, blk):
    k, n = tabT.shape
    return pl.pallas_call(
        _tr_body,
        grid=(pl.cdiv(n, blk),),
        in_specs=[pl.BlockSpec((k, blk), lambda b: (0, b))],
        out_specs=pl.BlockSpec((blk, k), lambda b: (b, 0)),
        out_shape=jax.ShapeDtypeStruct((n, k), jnp.float32),
    )(tabT)


def _make_gather_kernel(batch, emb_k):
    per_w = batch // N_WORKERS
    ch = per_w // 2
    mesh = plsc.VectorSubcoreMesh(core_axis_name="c", subcore_axis_name="s")

    @functools.partial(
        pl.kernel,
        mesh=mesh,
        compiler_params=pltpu.CompilerParams(needs_layout_passes=False),
        out_type=[
            jax.ShapeDtypeStruct((batch, emb_k), jnp.float32),
            jax.ShapeDtypeStruct((batch, emb_k), jnp.float32),
        ],
        scratch_types=[
            pltpu.VMEM((per_w,), jnp.int32),
            pltpu.VMEM((per_w,), jnp.int32),
            pltpu.VMEM((ch, emb_k), jnp.float32),
            pltpu.VMEM((ch, emb_k), jnp.float32),
            pltpu.SemaphoreType.DMA,
            pltpu.SemaphoreType.DMA,
        ],
    )
    def gather_kernel(uidx_hbm, iidx_hbm, utab_hbm, itab_hbm,
                      uout_hbm, iout_hbm,
                      uidx_v, iidx_v, buf0, buf1, sem0, sem1):
        wid = lax.axis_index("s") * 2 + lax.axis_index("c")
        base = wid * per_w
        pltpu.sync_copy(uidx_hbm.at[pl.ds(base, per_w)], uidx_v)
        pltpu.sync_copy(iidx_hbm.at[pl.ds(base, per_w)], iidx_v)
        lane = lax.iota(jnp.int32, 16)

        def fire(tab, idx_ref, idx_off, buf, sem):
            def group(g, c):
                iv = idx_ref[pl.ds(idx_off + g * 16, 16)]
                for l in range(16):
                    s = jnp.sum(jnp.where(lane == l, iv, 0))
                    pltpu.async_copy(tab.at[pl.ds(s, 1)],
                                     buf.at[pl.ds(g * 16 + l, 1)], sem)
                return c
            lax.fori_loop(0, ch // 16, group, 0)

        def drain_store(buf, sem, out, out_off):
            pltpu.make_async_copy(out.at[pl.ds(out_off, ch)], buf,
                                  sem).wait()
            pltpu.sync_copy(buf, out.at[pl.ds(out_off, ch)])

        batches = [
            (utab_hbm, uidx_v, 0, uout_hbm, base),
            (utab_hbm, uidx_v, ch, uout_hbm, base + ch),
            (itab_hbm, iidx_v, 0, iout_hbm, base),
            (itab_hbm, iidx_v, ch, iout_hbm, base + ch),
        ]
        bufs = (buf0, buf1)
        sems = (sem0, sem1)
        for b in range(4):
            tab, idx_ref, idx_off, out, out_off = batches[b]
            fire(tab, idx_ref, idx_off, bufs[b % 2], sems[b % 2])
            if b >= 1:
                ptab, pidx, pioff, pout, pooff = batches[b - 1]
                drain_store(bufs[(b - 1) % 2], sems[(b - 1) % 2],
                            pout, pooff)
        tab, idx_ref, idx_off, out, out_off = batches[3]
        drain_store(bufs[3 % 2], sems[3 % 2], out, out_off)

    return gather_kernel


def _mlp_body(u_ref, i_ref, w1_ref, b1_ref, w2_ref, out_ref):
    u = u_ref[...]
    it = i_ref[...]
    w1 = w1_ref[...]
    wa = w1[:, :EMB_K]
    wb = w1[:, EMB_K:]
    dn = (((1,), (1,)), ((), ()))
    h = lax.dot_general(u, wa, dn, preferred_element_type=jnp.float32)
    h = h + lax.dot_general(it, wb, dn, preferred_element_type=jnp.float32)
    h = jnp.maximum(h + b1_ref[...], 0.0)
    out_ref[...] = lax.dot_general(h, w2_ref[...], dn,
                                   preferred_element_type=jnp.float32)


def _mlp(u, it, W1, b1, W2, blk):
    batch = u.shape[0]
    grid = (batch // blk,)
    return pl.pallas_call(
        _mlp_body,
        grid=grid,
        in_specs=[
            pl.BlockSpec((blk, EMB_K), lambda b: (b, 0)),
            pl.BlockSpec((blk, EMB_K), lambda b: (b, 0)),
            pl.BlockSpec((EMB_K, 2 * EMB_K), lambda b: (0, 0)),
            pl.BlockSpec((1, EMB_K), lambda b: (0, 0)),
            pl.BlockSpec((1, EMB_K), lambda b: (0, 0)),
        ],
        out_specs=pl.BlockSpec((blk, 1), lambda b: (b, 0)),
        out_shape=jax.ShapeDtypeStruct((batch, 1), jnp.float32),
    )(u, it, W1, b1.reshape(1, EMB_K), W2)


def kernel(x, user_table, item_table, W1, b1, W2):
    batch = x.shape[0]
    emb_k = user_table.shape[1]
    uidx = x[:, 0]
    iidx = x[:, 1]
    utab = _to_row_major(user_table.T, blk=2048)
    itab = _to_row_major(item_table.T, blk=2048)
    gk = _make_gather_kernel(batch, emb_k)
    user_embed, item_embed = gk(uidx, iidx, utab, itab)
    out = _mlp(user_embed, item_embed, W1, b1, W2, blk=2048)
    return (out, user_embed, item_embed)


# speedup vs baseline: 1.2096x; 1.2096x over previous
import functools

import jax
import jax.numpy as jnp
from jax import lax
from jax.experimental import pallas as pl
from jax.experimental.pallas import tpu as pltpu
from jax.experimental.pallas import tpu_sc as plsc

EMB_K = 64
N_WORKERS = 32
LANES = 128
NBLK_FULL = 7812
BPW = 245
RING = 32
MAX_OUT = 24


def _make_gather_kernel(batch, emb_k, n_rows):
    tail_w = n_rows - NBLK_FULL * LANES
    tail_c0 = NBLK_FULL * LANES
    mesh = plsc.VectorSubcoreMesh(core_axis_name="c", subcore_axis_name="s")

    @functools.partial(
        pl.kernel,
        mesh=mesh,
        compiler_params=pltpu.CompilerParams(needs_layout_passes=False),
        out_type=[
            jax.ShapeDtypeStruct((batch, emb_k), jnp.float32),
            jax.ShapeDtypeStruct((batch, emb_k), jnp.float32),
        ],
        scratch_types=[
            pltpu.VMEM((batch,), jnp.int32),
            pltpu.VMEM((batch,), jnp.int32),
            pltpu.VMEM((batch,), jnp.int32),
            pltpu.VMEM((batch,), jnp.int32),
            pltpu.VMEM((256,), jnp.int32),
            pltpu.VMEM((256,), jnp.int32),
            pltpu.VMEM((2, emb_k, LANES), jnp.float32),
            pltpu.VMEM((emb_k, tail_w), jnp.float32),
            pltpu.VMEM((RING, emb_k), jnp.float32),
            pltpu.SemaphoreType.DMA,
            pltpu.SemaphoreType.DMA,
        ],
    )
    def gather_kernel(usv_hbm, uov_hbm, isv_hbm, iov_hbm,
                      uoffs_hbm, ioffs_hbm, utabT_hbm, itabT_hbm,
                      uout_hbm, iout_hbm,
                      usv, uov, isv, iov, uoffs, ioffs,
                      bbuf, tbuf, slots, semb, semo):
        wid = lax.axis_index("s") * 2 + lax.axis_index("c")
        c0g = wid * BPW
        pltpu.sync_copy(usv_hbm, usv)
        pltpu.sync_copy(uov_hbm, uov)
        pltpu.sync_copy(isv_hbm, isv)
        pltpu.sync_copy(iov_hbm, iov)
        pltpu.sync_copy(uoffs_hbm.at[wid], uoffs)
        pltpu.sync_copy(ioffs_hbm.at[wid], ioffs)
        lane = lax.iota(jnp.int32, 16)

        def extract(ref, pos):
            base = (pos // 16) * 16
            v = ref[pl.ds(base, 16)]
            return jnp.sum(jnp.where(lane == pos - base, v, 0))

        def wait_block():
            pltpu.make_async_copy(
                utabT_hbm.at[:, pl.ds(0, LANES)], bbuf.at[0], semb).wait()

        def wait_row():
            pltpu.make_async_copy(
                uout_hbm.at[pl.ds(0, 1)], slots.at[pl.ds(0, 1)], semo).wait()

        def do_hit(sval, sord, out, h, n, gather_col):
            v = extract(sval, h)
            p = extract(sord, h)
            slot = n % RING
            for q in range(4):
                col = gather_col(v, q)
                slots[slot, pl.ds(q * 16, 16)] = col
            pltpu.async_copy(slots.at[pl.ds(slot, 1)], out.at[pl.ds(p, 1)],
                             semo)
            n = n + 1

            @pl.when(n > MAX_OUT)
            def _():
                wait_row()
            return n

        def do_table(tabT, sval, sord, offs, out, n):
            def start(c):
                cg = jnp.minimum(c0g + c, NBLK_FULL - 1)
                pltpu.async_copy(tabT.at[:, pl.ds(cg * LANES, LANES)],
                                 bbuf.at[c % 2], semb)

            start(0)

            def block_body(c, n):
                start(c + 1)
                wait_block()
                hs = extract(offs, c)
                he = extract(offs, c + 1)
                he = jnp.where(c0g + c < NBLK_FULL, he, hs)
                par = (c % 2) + lane * 0

                def col_from_block(v, q):
                    kvec = q * 16 + lane
                    lvec = (v % LANES) + lane * 0
                    return plsc.load_gather(bbuf, [par, kvec, lvec])

                def hit_body(h, n):
                    return do_hit(sval, sord, out, h, n, col_from_block)

                return lax.fori_loop(hs, he, hit_body, n)

            n = lax.fori_loop(0, BPW, block_body, n)
            wait_block()

            @pl.when(wid == N_WORKERS - 1)
            def _():
                pltpu.sync_copy(tabT.at[:, pl.ds(tail_c0, tail_w)], tbuf)

            def tail_col(v, q):
                kvec = q * 16 + lane
                lvec = (v - tail_c0) + lane * 0
                return plsc.load_gather(tbuf, [kvec, lvec])

            def tail_hit(h, n):
                return do_hit(sval, sord, out, h, n, tail_col)

            is_tail_owner = wid == N_WORKERS - 1
            t_idx = jnp.where(is_tail_owner, NBLK_FULL - c0g, 0)
            hs = extract(offs, t_idx)
            he = extract(offs, t_idx + 1)
            hs = jnp.where(is_tail_owner, hs, 0)
            he = jnp.where(is_tail_owner, he, 0)
            return lax.fori_loop(hs, he, tail_hit, n)

        n = do_table(utabT_hbm, usv, uov, uoffs, uout_hbm, 0)
        n = do_table(itabT_hbm, isv, iov, ioffs, iout_hbm, n)

        def drain(_, c):
            wait_row()
            return c
        lax.fori_loop(0, jnp.minimum(n, MAX_OUT), drain, 0)

    return gather_kernel


def _mlp_body(u_ref, i_ref, w1_ref, b1_ref, w2_ref, out_ref):
    u = u_ref[...]
    it = i_ref[...]
    w1 = w1_ref[...]
    wa = w1[:, :EMB_K]
    wb = w1[:, EMB_K:]
    dn = (((1,), (1,)), ((), ()))
    h = lax.dot_general(u, wa, dn, preferred_element_type=jnp.float32)
    h = h + lax.dot_general(it, wb, dn, preferred_element_type=jnp.float32)
    h = jnp.maximum(h + b1_ref[...], 0.0)
    out_ref[...] = lax.dot_general(h, w2_ref[...], dn,
                                   preferred_element_type=jnp.float32)


def _mlp(u, it, W1, b1, W2, blk):
    batch = u.shape[0]
    return pl.pallas_call(
        _mlp_body,
        grid=(batch // blk,),
        in_specs=[
            pl.BlockSpec((blk, EMB_K), lambda b: (b, 0)),
            pl.BlockSpec((blk, EMB_K), lambda b: (b, 0)),
            pl.BlockSpec((EMB_K, 2 * EMB_K), lambda b: (0, 0)),
            pl.BlockSpec((1, EMB_K), lambda b: (0, 0)),
            pl.BlockSpec((1, EMB_K), lambda b: (0, 0)),
        ],
        out_specs=pl.BlockSpec((blk, 1), lambda b: (b, 0)),
        out_shape=jax.ShapeDtypeStruct((batch, 1), jnp.float32),
    )(u, it, W1, b1.reshape(1, EMB_K), W2)


def _prep(idx, batch):
    pos = jnp.arange(batch, dtype=jnp.int32)
    sval, sord = lax.sort_key_val(idx, pos)
    t = jnp.arange(256, dtype=jnp.int32)[None, :]
    w = jnp.arange(N_WORKERS, dtype=jnp.int32)[:, None]
    bounds = (w * BPW + t) * LANES
    offs = jnp.searchsorted(sval, bounds.reshape(-1),
                            side="left").astype(jnp.int32)
    return sval, sord, offs.reshape(N_WORKERS, 256)


def kernel(x, user_table, item_table, W1, b1, W2):
    batch = x.shape[0]
    emb_k = user_table.shape[1]
    n_rows = user_table.shape[0]
    usv, uov, uoffs = _prep(x[:, 0], batch)
    isv, iov, ioffs = _prep(x[:, 1], batch)
    gk = _make_gather_kernel(batch, emb_k, n_rows)
    user_embed, item_embed = gk(usv, uov, isv, iov, uoffs, ioffs,
                                user_table.T, item_table.T)
    out = _mlp(user_embed, item_embed, W1, b1, W2, blk=2048)
    return (out, user_embed, item_embed)


# speedup vs baseline: 3.5234x; 2.9130x over previous
import functools

import jax
import jax.numpy as jnp
from jax import lax
from jax.experimental import pallas as pl
from jax.experimental.pallas import tpu as pltpu
from jax.experimental.pallas import tpu_sc as plsc

EMB_K = 64
N_WORKERS = 32
CW = 512
NCH_FULL = 1953
CPW = 62
TAIL_C0 = NCH_FULL * CW
RING = 32
MAX_OUT = 24


def _make_gather_kernel(batch, emb_k, n_rows):
    tail_w = n_rows - TAIL_C0
    mesh = plsc.VectorSubcoreMesh(core_axis_name="c", subcore_axis_name="s")

    @functools.partial(
        pl.kernel,
        mesh=mesh,
        compiler_params=pltpu.CompilerParams(needs_layout_passes=False),
        out_type=[
            jax.ShapeDtypeStruct((batch, emb_k), jnp.float32),
            jax.ShapeDtypeStruct((batch, emb_k), jnp.float32),
        ],
        scratch_types=[
            pltpu.VMEM((batch,), jnp.int32),
            pltpu.VMEM((batch,), jnp.int32),
            pltpu.VMEM((40,), jnp.int32),
            pltpu.VMEM((2, emb_k, CW), jnp.float32),
            pltpu.VMEM((emb_k, n_rows - TAIL_C0), jnp.float32),
            pltpu.VMEM((RING, emb_k), jnp.float32),
            pltpu.SemaphoreType.DMA,
            pltpu.SemaphoreType.DMA,
        ],
    )
    def gather_kernel(usv_hbm, uov_hbm, isv_hbm, iov_hbm,
                      uwb_hbm, iwb_hbm, utabT_hbm, itabT_hbm,
                      uout_hbm, iout_hbm,
                      sval, sord, wb, bbuf, tbuf, slots, semb, semo):
        wid = lax.axis_index("s") * 2 + lax.axis_index("c")
        lane = lax.iota(jnp.int32, 16)

        def extract(ref, pos):
            base = (pos // 16) * 16
            v = ref[pl.ds(base, 16)]
            return jnp.sum(jnp.where(lane == pos - base, v, 0))

        def wait_chunk():
            pltpu.make_async_copy(
                utabT_hbm.at[:, pl.ds(0, CW)], bbuf.at[0], semb).wait()

        def wait_row():
            pltpu.make_async_copy(
                uout_hbm.at[pl.ds(0, 1)], slots.at[pl.ds(0, 1)], semo).wait()

        def do_hit(out, h, n, gather_col):
            v = extract(sval, h)
            p = extract(sord, h)
            slot = n % RING
            for q in range(4):
                col = gather_col(v, q)
                slots[slot, pl.ds(q * 16, 16)] = col
            pltpu.async_copy(slots.at[pl.ds(slot, 1)], out.at[pl.ds(p, 1)],
                             semo)
            n = n + 1

            @pl.when(n > MAX_OUT)
            def _():
                wait_row()
            return n

        def do_table(tabT, sv_hbm, so_hbm, wb_hbm, out, n):
            pltpu.sync_copy(sv_hbm, sval)
            pltpu.sync_copy(so_hbm, sord)
            pltpu.sync_copy(wb_hbm, wb)
            he_w = extract(wb, wid + 1)
            ptr0 = extract(wb, wid)

            def start(c):
                cg = jnp.minimum(wid * CPW + c, NCH_FULL - 1)
                pltpu.async_copy(tabT.at[:, pl.ds(cg * CW, CW)],
                                 bbuf.at[c % 2], semb)

            start(0)

            def chunk_body(c, carry):
                ptr, n = carry
                start(c + 1)
                wait_chunk()
                limit = jnp.minimum((wid * CPW + c + 1) * CW, TAIL_C0)
                par = (c % 2) + lane * 0

                def col_from_chunk(v, q):
                    kvec = q * 16 + lane
                    lvec = (v % CW) + lane * 0
                    return plsc.load_gather(bbuf, [par, kvec, lvec])

                def cond(s):
                    p_, _ = s
                    return jnp.logical_and(p_ < he_w,
                                           extract(sval, p_) < limit)

                def body(s):
                    p_, n_ = s
                    n_ = do_hit(out, p_, n_, col_from_chunk)
                    return p_ + 1, n_

                return lax.while_loop(cond, body, (ptr, n))

            ptr, n = lax.fori_loop(0, CPW, chunk_body, (ptr0, n))
            wait_chunk()

            @pl.when(wid == N_WORKERS - 1)
            def _():
                pltpu.sync_copy(tabT.at[:, pl.ds(TAIL_C0, tail_w)], tbuf)

            def tail_col(v, q):
                kvec = q * 16 + lane
                lvec = (v - TAIL_C0) + lane * 0
                return plsc.load_gather(tbuf, [kvec, lvec])

            def tail_cond(s):
                p_, _ = s
                return p_ < he_w

            def tail_body(s):
                p_, n_ = s
                n_ = do_hit(out, p_, n_, tail_col)
                return p_ + 1, n_

            ptr, n = lax.while_loop(tail_cond, tail_body, (ptr, n))
            return n

        n = do_table(utabT_hbm, usv_hbm, uov_hbm, uwb_hbm, uout_hbm, 0)
        n = do_table(itabT_hbm, isv_hbm, iov_hbm, iwb_hbm, iout_hbm, n)

        def drain(_, c):
            wait_row()
            return c
        lax.fori_loop(0, jnp.minimum(n, MAX_OUT), drain, 0)

    return gather_kernel


def _mlp_body(u_ref, i_ref, w1_ref, b1_ref, w2_ref, out_ref):
    u = u_ref[...]
    it = i_ref[...]
    w1 = w1_ref[...]
    wa = w1[:, :EMB_K]
    wb = w1[:, EMB_K:]
    dn = (((1,), (1,)), ((), ()))
    h = lax.dot_general(u, wa, dn, preferred_element_type=jnp.float32)
    h = h + lax.dot_general(it, wb, dn, preferred_element_type=jnp.float32)
    h = jnp.maximum(h + b1_ref[...], 0.0)
    out_ref[...] = lax.dot_general(h, w2_ref[...], dn,
                                   preferred_element_type=jnp.float32)


def _mlp(u, it, W1, b1, W2, blk):
    batch = u.shape[0]
    return pl.pallas_call(
        _mlp_body,
        grid=(batch // blk,),
        in_specs=[
            pl.BlockSpec((blk, EMB_K), lambda b: (b, 0)),
            pl.BlockSpec((blk, EMB_K), lambda b: (b, 0)),
            pl.BlockSpec((EMB_K, 2 * EMB_K), lambda b: (0, 0)),
            pl.BlockSpec((1, EMB_K), lambda b: (0, 0)),
            pl.BlockSpec((1, EMB_K), lambda b: (0, 0)),
        ],
        out_specs=pl.BlockSpec((blk, 1), lambda b: (b, 0)),
        out_shape=jax.ShapeDtypeStruct((batch, 1), jnp.float32),
    )(u, it, W1, b1.reshape(1, EMB_K), W2)


def _prep(idx, batch):
    pos = jnp.arange(batch, dtype=jnp.int32)
    sval, sord = lax.sort_key_val(idx, pos)
    bounds = jnp.arange(33, dtype=jnp.int32) * (CPW * CW)
    wb = jnp.searchsorted(sval, bounds, side="left").astype(jnp.int32)
    wb = jnp.pad(wb, (0, 7))
    return sval, sord, wb


def kernel(x, user_table, item_table, W1, b1, W2):
    batch = x.shape[0]
    emb_k = user_table.shape[1]
    n_rows = user_table.shape[0]
    usv, uov, uwb = _prep(x[:, 0], batch)
    isv, iov, iwb = _prep(x[:, 1], batch)
    gk = _make_gather_kernel(batch, emb_k, n_rows)
    user_embed, item_embed = gk(usv, uov, isv, iov, uwb, iwb,
                                user_table.T, item_table.T)
    out = _mlp(user_embed, item_embed, W1, b1, W2, blk=2048)
    return (out, user_embed, item_embed)


# speedup vs baseline: 3.7309x; 1.0589x over previous
import functools

import jax
import jax.numpy as jnp
from jax import lax
from jax.experimental import pallas as pl
from jax.experimental.pallas import tpu as pltpu
from jax.experimental.pallas import tpu_sc as plsc

EMB_K = 64
N_WORKERS = 32
CW = 384
NCH_FULL = 2604
CPW = 82
TAIL_C0 = NCH_FULL * CW
RING = 32
MAX_OUT = 24


def _make_gather_kernel(batch, emb_k, n_rows):
    tail_w = n_rows - TAIL_C0
    mesh = plsc.VectorSubcoreMesh(core_axis_name="c", subcore_axis_name="s")

    @functools.partial(
        pl.kernel,
        mesh=mesh,
        compiler_params=pltpu.CompilerParams(needs_layout_passes=False),
        out_type=[
            jax.ShapeDtypeStruct((batch, emb_k), jnp.float32),
            jax.ShapeDtypeStruct((batch, emb_k), jnp.float32),
        ],
        scratch_types=[
            pltpu.VMEM((batch,), jnp.int32),
            pltpu.VMEM((batch,), jnp.int32),
            pltpu.VMEM((40,), jnp.int32),
            pltpu.VMEM((3, emb_k, CW), jnp.float32),
            pltpu.VMEM((emb_k, n_rows - TAIL_C0), jnp.float32),
            pltpu.VMEM((RING, emb_k), jnp.float32),
            pltpu.SemaphoreType.DMA,
            pltpu.SemaphoreType.DMA,
        ],
    )
    def gather_kernel(usv_hbm, uov_hbm, isv_hbm, iov_hbm,
                      uwb_hbm, iwb_hbm, utabT_hbm, itabT_hbm,
                      uout_hbm, iout_hbm,
                      sval, sord, wb, bbuf, tbuf, slots, semb, semo):
        wid = lax.axis_index("s") * 2 + lax.axis_index("c")
        lane = lax.iota(jnp.int32, 16)

        def extract(ref, pos):
            base = (pos // 16) * 16
            v = ref[pl.ds(base, 16)]
            return jnp.sum(jnp.where(lane == pos - base, v, 0))

        def wait_chunk():
            pltpu.make_async_copy(
                utabT_hbm.at[:, pl.ds(0, CW)], bbuf.at[0], semb).wait()

        def wait_row():
            pltpu.make_async_copy(
                uout_hbm.at[pl.ds(0, 1)], slots.at[pl.ds(0, 1)], semo).wait()

        def do_hit(out, h, n, gather_col):
            v = extract(sval, h)
            p = extract(sord, h)
            slot = n % RING
            for q in range(4):
                col = gather_col(v, q)
                slots[slot, pl.ds(q * 16, 16)] = col
            pltpu.async_copy(slots.at[pl.ds(slot, 1)], out.at[pl.ds(p, 1)],
                             semo)
            n = n + 1

            @pl.when(n > MAX_OUT)
            def _():
                wait_row()
            return n

        def do_table(tabT, sv_hbm, so_hbm, wb_hbm, out, n):
            pltpu.sync_copy(sv_hbm, sval)
            pltpu.sync_copy(so_hbm, sord)
            pltpu.sync_copy(wb_hbm, wb)
            he_w = extract(wb, wid + 1)
            ptr0 = extract(wb, wid)

            def start(c):
                cg = jnp.minimum(wid * CPW + c, NCH_FULL - 1)
                pltpu.async_copy(tabT.at[:, pl.ds(cg * CW, CW)],
                                 bbuf.at[c % 3], semb)

            start(0)
            start(1)

            def chunk_body(c, carry):
                ptr, n = carry
                start(c + 2)
                wait_chunk()
                limit = jnp.minimum((wid * CPW + c + 1) * CW, TAIL_C0)
                par = (c % 3) + lane * 0

                def col_from_chunk(v, q):
                    kvec = q * 16 + lane
                    lvec = (v % CW) + lane * 0
                    return plsc.load_gather(bbuf, [par, kvec, lvec])

                def cond(s):
                    p_, _ = s
                    return jnp.logical_and(p_ < he_w,
                                           extract(sval, p_) < limit)

                def body(s):
                    p_, n_ = s
                    n_ = do_hit(out, p_, n_, col_from_chunk)
                    return p_ + 1, n_

                return lax.while_loop(cond, body, (ptr, n))

            ptr, n = lax.fori_loop(0, CPW, chunk_body, (ptr0, n))
            wait_chunk()
            wait_chunk()

            @pl.when(wid == N_WORKERS - 1)
            def _():
                pltpu.sync_copy(tabT.at[:, pl.ds(TAIL_C0, tail_w)], tbuf)

            def tail_col(v, q):
                kvec = q * 16 + lane
                lvec = (v - TAIL_C0) + lane * 0
                return plsc.load_gather(tbuf, [kvec, lvec])

            def tail_cond(s):
                p_, _ = s
                return p_ < he_w

            def tail_body(s):
                p_, n_ = s
                n_ = do_hit(out, p_, n_, tail_col)
                return p_ + 1, n_

            ptr, n = lax.while_loop(tail_cond, tail_body, (ptr, n))
            return n

        n = do_table(utabT_hbm, usv_hbm, uov_hbm, uwb_hbm, uout_hbm, 0)
        n = do_table(itabT_hbm, isv_hbm, iov_hbm, iwb_hbm, iout_hbm, n)

        def drain(_, c):
            wait_row()
            return c
        lax.fori_loop(0, jnp.minimum(n, MAX_OUT), drain, 0)

    return gather_kernel


def _mlp_body(u_ref, i_ref, w1_ref, b1_ref, w2_ref, out_ref):
    u = u_ref[...]
    it = i_ref[...]
    w1 = w1_ref[...]
    wa = w1[:, :EMB_K]
    wb = w1[:, EMB_K:]
    dn = (((1,), (1,)), ((), ()))
    h = lax.dot_general(u, wa, dn, preferred_element_type=jnp.float32)
    h = h + lax.dot_general(it, wb, dn, preferred_element_type=jnp.float32)
    h = jnp.maximum(h + b1_ref[...], 0.0)
    out_ref[...] = lax.dot_general(h, w2_ref[...], dn,
                                   preferred_element_type=jnp.float32)


def _mlp(u, it, W1, b1, W2, blk):
    batch = u.shape[0]
    return pl.pallas_call(
        _mlp_body,
        grid=(batch // blk,),
        in_specs=[
            pl.BlockSpec((blk, EMB_K), lambda b: (b, 0)),
            pl.BlockSpec((blk, EMB_K), lambda b: (b, 0)),
            pl.BlockSpec((EMB_K, 2 * EMB_K), lambda b: (0, 0)),
            pl.BlockSpec((1, EMB_K), lambda b: (0, 0)),
            pl.BlockSpec((1, EMB_K), lambda b: (0, 0)),
        ],
        out_specs=pl.BlockSpec((blk, 1), lambda b: (b, 0)),
        out_shape=jax.ShapeDtypeStruct((batch, 1), jnp.float32),
    )(u, it, W1, b1.reshape(1, EMB_K), W2)


def _prep(idx, batch):
    pos = jnp.arange(batch, dtype=jnp.int32)
    sval, sord = lax.sort_key_val(idx, pos)
    bounds = jnp.arange(33, dtype=jnp.int32) * (CPW * CW)
    wb = jnp.searchsorted(sval, bounds, side="left").astype(jnp.int32)
    wb = jnp.pad(wb, (0, 7))
    return sval, sord, wb


def kernel(x, user_table, item_table, W1, b1, W2):
    batch = x.shape[0]
    emb_k = user_table.shape[1]
    n_rows = user_table.shape[0]
    usv, uov, uwb = _prep(x[:, 0], batch)
    isv, iov, iwb = _prep(x[:, 1], batch)
    gk = _make_gather_kernel(batch, emb_k, n_rows)
    user_embed, item_embed = gk(usv, uov, isv, iov, uwb, iwb,
                                user_table.T, item_table.T)
    out = _mlp(user_embed, item_embed, W1, b1, W2, blk=2048)
    return (out, user_embed, item_embed)
